# baseline (speedup 1.0000x reference)
import functools

import jax
import jax.numpy as jnp
from jax import lax
from jax.experimental import pallas as pl
from jax.experimental.pallas import tpu as pltpu
from jax.experimental.pallas import tpu_sc as plsc

NC = 2
NS = 16
NW = NC * NS
CHUNK = 128
CP = 128


def _round_up(v, m):
    return (v + m - 1) // m * m


def _make_sc_hist(EP, NP):
    CH = EP // (NS * CHUNK)
    rows_per_tec = NP // NS
    mesh = plsc.VectorSubcoreMesh(core_axis_name="c", subcore_axis_name="s")

    @functools.partial(
        pl.kernel,
        out_type=jax.ShapeDtypeStruct((NP,), jnp.float32),
        mesh=mesh,
        scratch_types=[
            pltpu.VMEM((CH, CHUNK), jnp.int32),
            pltpu.VMEM((CHUNK,), jnp.float32),
            pltpu.VMEM((rows_per_tec,), jnp.float32),
            pltpu.VMEM_SHARED((NP,), jnp.float32),
        ],
    )
    def hist(dst2d, deg_out, idx_buf, ones_buf, row_buf, accum):
        c = lax.axis_index("c")
        t = lax.axis_index("s")

        @pl.when(c == 0)
        def _():
            @pl.loop(0, rows_per_tec // 16)
            def _(i):
                row_buf[pl.ds(i * 16, 16)] = jnp.zeros((16,), jnp.float32)

            @pl.loop(0, CHUNK // 16)
            def _(i):
                ones_buf[pl.ds(i * 16, 16)] = jnp.ones((16,), jnp.float32)

            pltpu.sync_copy(row_buf, accum.at[pl.ds(t * rows_per_tec,
                                                    rows_per_tec)])
            pltpu.sync_copy(dst2d.at[pl.ds(t * CH, CH)], idx_buf)
            plsc.subcore_barrier()

            @pl.loop(0, CH)
            def _(j):
                pltpu.sync_copy(ones_buf, accum.at[idx_buf.at[j]], add=True)

            plsc.subcore_barrier()
            pltpu.sync_copy(accum.at[pl.ds(t * rows_per_tec, rows_per_tec)],
                            row_buf)
            pltpu.sync_copy(row_buf,
                            deg_out.at[pl.ds(t * rows_per_tec, rows_per_tec)])

    return hist


def _make_sc_part(EP, NPQ):
    RCAP = EP // NW
    NV = RCAP // 16
    NB = 2 * NC
    B1 = RCAP + 16
    mesh = plsc.VectorSubcoreMesh(core_axis_name="c", subcore_axis_name="s")

    @functools.partial(
        pl.kernel,
        compiler_params=pltpu.CompilerParams(needs_layout_passes=False),
        out_type=(jax.ShapeDtypeStruct((2 * NB * NW * RCAP,), jnp.int32),
                  jax.ShapeDtypeStruct((2 * NB * NW * RCAP,), jnp.int32),
                  jax.ShapeDtypeStruct((NB * NW * 16,), jnp.int32)),
        mesh=mesh,
        scratch_types=[
            pltpu.VMEM((RCAP,), jnp.int32),
            pltpu.VMEM((RCAP,), jnp.int32),
            pltpu.VMEM((2 * NB * B1,), jnp.int32),
            pltpu.VMEM((2 * NB * B1,), jnp.int32),
            pltpu.VMEM((16,), jnp.int32),
            pltpu.VMEM((32,), jnp.int32),
        ],
    )
    def part(src_flat, dst_flat, psrc, pdst, pcnt,
             src_in, dst_in, sbuf, dbuf, cbuf, tmp):
        c = lax.axis_index("c")
        t = lax.axis_index("s")
        w = c * NS + t
        base = w * RCAP

        pltpu.sync_copy(src_flat.at[pl.ds(base, RCAP)], src_in)
        pltpu.sync_copy(dst_flat.at[pl.ds(base, RCAP)], dst_in)

        parity = lax.iota(jnp.int32, 16) % 2

        @pl.loop(0, 2 * NB * B1 // 16)
        def _(i):
            sbuf[pl.ds(i * 16, 16)] = parity
            dbuf[pl.ds(i * 16, 16)] = 2 * NPQ + parity

        tmp[pl.ds(0, 16)] = jnp.zeros((16,), jnp.int32)

        def prefix(x):
            for k in (1, 2, 4, 8):
                tmp[pl.ds(16, 16)] = x
                x = x + tmp[pl.ds(16 - k, 16)]
            return x

        def body(i, carry):
            offs = carry
            sv = src_in[pl.ds(i * 16, 16)]
            dv = dst_in[pl.ds(i * 16, 16)]
            q = ((dv >= NPQ).astype(jnp.int32)
                 + (dv >= 2 * NPQ).astype(jnp.int32)
                 + (dv >= 3 * NPQ).astype(jnp.int32))
            dloc = dv - q * NPQ
            idx = jnp.zeros((16,), jnp.int32)
            ns = []
            for b in range(NB):
                mb = (q == b).astype(jnp.int32)
                cb = prefix(mb)
                idx = idx + mb * (b * B1 + offs[b] + cb - 1)
                ns.append(cb[15])
            plsc.store_scatter(sbuf, [2 * idx], 2 * sv)
            plsc.store_scatter(sbuf, [2 * idx + 1], 2 * sv + 1)
            plsc.store_scatter(dbuf, [2 * idx], 2 * dloc)
            plsc.store_scatter(dbuf, [2 * idx + 1], 2 * dloc + 1)
            return tuple(offs[b] + ns[b] for b in range(NB))

        offs = pl.loop(0, NV, init_carry=tuple(
            jnp.int32(0) for _ in range(NB)))(body)

        for b in range(NB):
            pltpu.sync_copy(sbuf.at[pl.ds(2 * b * B1, 2 * RCAP)],
                            psrc.at[pl.ds((b * NW + w) * 2 * RCAP, 2 * RCAP)])
            pltpu.sync_copy(dbuf.at[pl.ds(2 * b * B1, 2 * RCAP)],
                            pdst.at[pl.ds((b * NW + w) * 2 * RCAP, 2 * RCAP)])
            cbuf[...] = jnp.full((16,), 1, jnp.int32) * offs[b]
            pltpu.sync_copy(cbuf, pcnt.at[pl.ds((b * NW + w) * 16, 16)])

    return part


def _make_sc_agg(EP, NP, D):
    NB = 2 * NC
    NPQ = NP // NB
    ACCX = 2 * NPQ + CP
    RCAP = EP // NW
    RCHX = 2 * RCAP // CP
    wb_half = 2 * NPQ // NS
    DH = D // 2
    mesh = plsc.VectorSubcoreMesh(core_axis_name="c", subcore_axis_name="s")

    @functools.partial(
        pl.kernel,
        out_type=jax.ShapeDtypeStruct((2 * NP, DH), jnp.float32),
        mesh=mesh,
        scratch_types=[
            pltpu.VMEM((RCHX, CP), jnp.int32),
            pltpu.VMEM((RCHX, CP), jnp.int32),
            pltpu.VMEM((16,), jnp.int32),
            pltpu.VMEM((CP, DH), jnp.float32),
            pltpu.VMEM((CP, DH), jnp.float32),
            pltpu.VMEM_SHARED((ACCX, DH), jnp.float32),
            pltpu.SemaphoreType.DMA,
            pltpu.SemaphoreType.DMA,
        ],
    )
    def agg(g_hbm, psrc, pdst, pcnt, zrows, s_out,
            src_b, dst_b, cnt_b, buf_a, buf_b, accum, sem_a, sem_b):
        c = lax.axis_index("c")
        t = lax.axis_index("s")

        def start(j, buf, sem):
            pltpu.async_copy(g_hbm.at[src_b.at[j]], buf, sem)

        def wait(buf, sem):
            pltpu.make_async_copy(g_hbm.at[pl.ds(0, CP)], buf, sem).wait()

        def scat(j, buf):
            pltpu.sync_copy(buf, accum.at[dst_b.at[j]], add=True)

        for p in range(NB // NC):
            b = (NB // NC) * c + p
            pltpu.sync_copy(zrows, buf_a)
            zb = t * wb_half
            for q in range(wb_half // CP):
                pltpu.sync_copy(buf_a, accum.at[pl.ds(zb + q * CP, CP)])
            if wb_half % CP:
                pltpu.sync_copy(buf_a.at[pl.ds(0, wb_half % CP)],
                                accum.at[pl.ds(zb + (wb_half // CP) * CP,
                                               wb_half % CP)])
            plsc.subcore_barrier()

            for r in range(NW // NS):
                reg = (NW // NS) * t + r
                rowbase = (b * NW + reg) * RCHX
                pltpu.sync_copy(psrc.at[pl.ds(rowbase, RCHX)], src_b)
                pltpu.sync_copy(pdst.at[pl.ds(rowbase, RCHX)], dst_b)
                pltpu.sync_copy(pcnt.at[pl.ds((b * NW + reg) * 16, 16)],
                                cnt_b)
                n = cnt_b[...][0]
                nch = jnp.maximum((2 * n + (CP - 1)) // CP + 1, 2) // 2 * 2

                start(0, buf_a, sem_a)
                start(1, buf_b, sem_b)

                @pl.loop(0, nch - 2, step=2)
                def _(j):
                    wait(buf_a, sem_a)
                    scat(j, buf_a)
                    start(j + 2, buf_a, sem_a)
                    wait(buf_b, sem_b)
                    scat(j + 1, buf_b)
                    start(j + 3, buf_b, sem_b)

                wait(buf_a, sem_a)
                scat(nch - 2, buf_a)
                wait(buf_b, sem_b)
                scat(nch - 1, buf_b)

            plsc.subcore_barrier()
            wb = t * wb_half
            nfull, rem = divmod(wb_half, CP)
            for q in range(nfull):
                sl = pl.ds(wb + q * CP, CP)
                pltpu.sync_copy(accum.at[sl], buf_a)
                pltpu.sync_copy(buf_a,
                                s_out.at[pl.ds(2 * b * NPQ + wb + q * CP,
                                               CP)])
            if rem:
                sl = pl.ds(wb + nfull * CP, rem)
                pltpu.sync_copy(accum.at[sl], buf_a.at[pl.ds(0, rem)])
                pltpu.sync_copy(
                    buf_a.at[pl.ds(0, rem)],
                    s_out.at[pl.ds(2 * b * NPQ + wb + nfull * CP, rem)])
            plsc.subcore_barrier()

    return agg


_BR = 256


def _make_tc_prep(N, NP, D):
    grid = NP // _BR

    def body(deg_ref, x_ref, inv_ref, g_ref):
        deg = deg_ref[...] + 1.0
        inv = lax.rsqrt(deg)
        inv_ref[...] = inv
        g_ref[...] = x_ref[...] * inv

    return pl.pallas_call(
        body,
        grid=(grid,),
        in_specs=[
            pl.BlockSpec((_BR, 1), lambda i: (i, 0)),
            pl.BlockSpec((_BR, D), lambda i: (i, 0)),
        ],
        out_specs=[
            pl.BlockSpec((_BR, 1), lambda i: (i, 0)),
            pl.BlockSpec((_BR, D), lambda i: (i, 0)),
        ],
        out_shape=[
            jax.ShapeDtypeStruct((NP, 1), jnp.float32),
            jax.ShapeDtypeStruct((NP, D), jnp.float32),
        ],
    )


def _make_tc_round(N, NP, D, final):
    grid = NP // _BR

    def body(s_ref, g_ref, inv_ref, w_ref, out_ref):
        inv = inv_ref[...]
        z = (s_ref[...] + g_ref[...]) * inv
        y = lax.dot_general(z, w_ref[...], (((1,), (1,)), ((), ())),
                            preferred_element_type=jnp.float32)
        y = jnp.maximum(y, 0.0)
        out_ref[...] = y if final else y * inv

    return pl.pallas_call(
        body,
        grid=(grid,),
        in_specs=[
            pl.BlockSpec((_BR, D), lambda i: (i, 0)),
            pl.BlockSpec((_BR, D), lambda i: (i, 0)),
            pl.BlockSpec((_BR, 1), lambda i: (i, 0)),
            pl.BlockSpec((D, D), lambda i: (0, 0)),
        ],
        out_specs=pl.BlockSpec((_BR, D), lambda i: (i, 0)),
        out_shape=jax.ShapeDtypeStruct((N if final else NP, D), jnp.float32),
    )


def kernel(x, edge_index, W0, W1, W2):
    N, D = x.shape
    E = edge_index.shape[1]
    NB = 2 * NC

    EP = _round_up(E, NW * CP * 8)
    NP = _round_up(N + 1, NS * CHUNK)
    NPQ = NP // NB
    RCAP = EP // NW
    RCHX = 2 * RCAP // CP

    src = edge_index[0]
    dst = edge_index[1]
    pad = EP - E
    src_p = jnp.concatenate([src, jnp.zeros((pad,), jnp.int32)])
    dst_p = jnp.concatenate([dst, jnp.full((pad,), N, jnp.int32)])
    dst2d = dst_p.reshape(EP // CHUNK, CHUNK)
    zrows = jnp.zeros((CP, D // 2), jnp.float32)

    deg_raw = _make_sc_hist(EP, NP)(dst2d)
    inv_col, g = _make_tc_prep(N, NP, D)(deg_raw.reshape(NP, 1), x)
    psrc, pdst, pcnt = _make_sc_part(EP, NPQ)(src_p, dst_p)
    psrc = psrc.reshape(NB * NW * RCHX, CP)
    pdst = pdst.reshape(NB * NW * RCHX, CP)

    sc_agg = _make_sc_agg(EP, NP, D)
    tc_mid = _make_tc_round(N, NP, D, final=False)
    tc_fin = _make_tc_round(N, NP, D, final=True)

    for r, W in enumerate((W0, W1, W2)):
        s_acc = sc_agg(g.reshape(2 * NP, D // 2), psrc, pdst, pcnt, zrows)
        s_acc = s_acc.reshape(NP, D)
        if r < 2:
            g = tc_mid(s_acc, g, inv_col, W)
        else:
            h = tc_fin(s_acc, g, inv_col, W)
    return h


# speedup vs baseline: 2.2238x; 2.2238x over previous
import functools

import jax
import jax.numpy as jnp
from jax import lax
from jax.experimental import pallas as pl
from jax.experimental.pallas import tpu as pltpu
from jax.experimental.pallas import tpu_sc as plsc

NC = 2
NS = 16
CHUNK = 128


def _round_up(v, m):
    return (v + m - 1) // m * m


def _make_sc_hist(EP, NP):
    CH = EP // (NS * CHUNK)
    rows_per_tec = NP // NS
    mesh = plsc.VectorSubcoreMesh(core_axis_name="c", subcore_axis_name="s")

    @functools.partial(
        pl.kernel,
        out_type=jax.ShapeDtypeStruct((NP,), jnp.float32),
        mesh=mesh,
        scratch_types=[
            pltpu.VMEM((CH, CHUNK), jnp.int32),
            pltpu.VMEM((CHUNK,), jnp.float32),
            pltpu.VMEM((rows_per_tec,), jnp.float32),
            pltpu.VMEM_SHARED((NP,), jnp.float32),
        ],
    )
    def hist(dst2d, deg_out, idx_buf, ones_buf, row_buf, accum):
        c = lax.axis_index("c")
        t = lax.axis_index("s")

        @pl.when(c == 0)
        def _():
            @pl.loop(0, rows_per_tec // 16)
            def _(i):
                row_buf[pl.ds(i * 16, 16)] = jnp.zeros((16,), jnp.float32)

            @pl.loop(0, CHUNK // 16)
            def _(i):
                ones_buf[pl.ds(i * 16, 16)] = jnp.ones((16,), jnp.float32)

            pltpu.sync_copy(row_buf, accum.at[pl.ds(t * rows_per_tec,
                                                    rows_per_tec)])
            pltpu.sync_copy(dst2d.at[pl.ds(t * CH, CH)], idx_buf)
            plsc.subcore_barrier()

            @pl.loop(0, CH)
            def _(j):
                pltpu.sync_copy(ones_buf, accum.at[idx_buf.at[j]], add=True)

            plsc.subcore_barrier()
            pltpu.sync_copy(accum.at[pl.ds(t * rows_per_tec, rows_per_tec)],
                            row_buf)
            pltpu.sync_copy(row_buf,
                            deg_out.at[pl.ds(t * rows_per_tec, rows_per_tec)])

    return hist


_GRP = 16


def _make_sc_agg(EP, NP, DH):
    CH = EP // (NS * CHUNK)
    NG = CH // _GRP
    rows_per_tec = NP // NS
    RB = rows_per_tec // CHUNK
    mesh = plsc.VectorSubcoreMesh(core_axis_name="c", subcore_axis_name="s")

    @functools.partial(
        pl.kernel,
        out_type=(jax.ShapeDtypeStruct((NP, DH), jnp.float32),
                  jax.ShapeDtypeStruct((NP, DH), jnp.float32)),
        mesh=mesh,
        scratch_types=[
            pltpu.VMEM((_GRP, CHUNK), jnp.int32),
            pltpu.VMEM((_GRP, CHUNK), jnp.int32),
            pltpu.VMEM((CHUNK, DH), jnp.float32),
            pltpu.VMEM((CHUNK, DH), jnp.float32),
            pltpu.VMEM_SHARED((NP, DH), jnp.float32),
            pltpu.SemaphoreType.DMA,
            pltpu.SemaphoreType.DMA,
        ],
    )
    def agg(gl, gr, src2d, dst2d, sl_out, sr_out,
            src_grp, dst_grp, buf_a, buf_b, accum, sem_a, sem_b):
        c = lax.axis_index("c")
        t = lax.axis_index("s")
        rbase = t * rows_per_tec

        def run(g_hbm, s_hbm):
            for j in range(RB):
                sl = pl.ds(rbase + j * CHUNK, CHUNK)
                pltpu.sync_copy(g_hbm.at[sl], buf_a)
                pltpu.sync_copy(buf_a, accum.at[sl])
            plsc.subcore_barrier()

            def start(j, buf, sem):
                pltpu.async_copy(g_hbm.at[src_grp.at[j]], buf, sem)

            def wait(buf, sem):
                pltpu.make_async_copy(g_hbm.at[pl.ds(0, CHUNK)], buf,
                                      sem).wait()

            def scat(j, buf):
                pltpu.sync_copy(buf, accum.at[dst_grp.at[j]], add=True)

            for g in range(NG):
                gsl = pl.ds(t * CH + g * _GRP, _GRP)
                pltpu.sync_copy(src2d.at[gsl], src_grp)
                pltpu.sync_copy(dst2d.at[gsl], dst_grp)
                start(0, buf_a, sem_a)

                @pl.loop(0, _GRP - 2, step=2)
                def _(j):
                    start(j + 1, buf_b, sem_b)
                    wait(buf_a, sem_a)
                    scat(j, buf_a)
                    start(j + 2, buf_a, sem_a)
                    wait(buf_b, sem_b)
                    scat(j + 1, buf_b)

                start(_GRP - 1, buf_b, sem_b)
                wait(buf_a, sem_a)
                scat(_GRP - 2, buf_a)
                wait(buf_b, sem_b)
                scat(_GRP - 1, buf_b)

            plsc.subcore_barrier()
            for j in range(RB):
                sl = pl.ds(rbase + j * CHUNK, CHUNK)
                pltpu.sync_copy(accum.at[sl], buf_a)
                pltpu.sync_copy(buf_a, s_hbm.at[sl])

        @pl.when(c == 0)
        def _():
            run(gl, sl_out)

        @pl.when(c == 1)
        def _():
            run(gr, sr_out)

    return agg


_BR = 256


def _make_tc_prep(N, NP, D):
    DH = D // 2
    grid = NP // _BR

    def body(deg_ref, x_ref, inv_ref, gl_ref, gr_ref):
        deg = deg_ref[...] + 1.0
        inv = lax.rsqrt(deg)
        inv_ref[...] = inv
        g = x_ref[...] * inv
        gl_ref[...] = g[:, :DH]
        gr_ref[...] = g[:, DH:]

    return pl.pallas_call(
        body,
        grid=(grid,),
        in_specs=[
            pl.BlockSpec((_BR, 1), lambda i: (i, 0)),
            pl.BlockSpec((_BR, D), lambda i: (i, 0)),
        ],
        out_specs=[
            pl.BlockSpec((_BR, 1), lambda i: (i, 0)),
            pl.BlockSpec((_BR, DH), lambda i: (i, 0)),
            pl.BlockSpec((_BR, DH), lambda i: (i, 0)),
        ],
        out_shape=[
            jax.ShapeDtypeStruct((NP, 1), jnp.float32),
            jax.ShapeDtypeStruct((NP, DH), jnp.float32),
            jax.ShapeDtypeStruct((NP, DH), jnp.float32),
        ],
    )


def _make_tc_round(N, NP, D, final):
    DH = D // 2
    grid = NP // _BR

    def body(sl_ref, sr_ref, inv_ref, w_ref, *out_refs):
        inv = inv_ref[...]
        z = jnp.concatenate([sl_ref[...], sr_ref[...]], axis=1) * inv
        y = lax.dot_general(z, w_ref[...], (((1,), (1,)), ((), ())),
                            preferred_element_type=jnp.float32)
        y = jnp.maximum(y, 0.0)
        if final:
            out_refs[0][...] = y
        else:
            g = y * inv
            out_refs[0][...] = g[:, :DH]
            out_refs[1][...] = g[:, DH:]

    if final:
        out_specs = [pl.BlockSpec((_BR, D), lambda i: (i, 0))]
        out_shape = [jax.ShapeDtypeStruct((N, D), jnp.float32)]
    else:
        out_specs = [pl.BlockSpec((_BR, DH), lambda i: (i, 0)),
                     pl.BlockSpec((_BR, DH), lambda i: (i, 0))]
        out_shape = [jax.ShapeDtypeStruct((NP, DH), jnp.float32),
                     jax.ShapeDtypeStruct((NP, DH), jnp.float32)]

    return pl.pallas_call(
        body,
        grid=(grid,),
        in_specs=[
            pl.BlockSpec((_BR, DH), lambda i: (i, 0)),
            pl.BlockSpec((_BR, DH), lambda i: (i, 0)),
            pl.BlockSpec((_BR, 1), lambda i: (i, 0)),
            pl.BlockSpec((D, D), lambda i: (0, 0)),
        ],
        out_specs=out_specs,
        out_shape=out_shape,
    )


def kernel(x, edge_index, W0, W1, W2):
    N, D = x.shape
    E = edge_index.shape[1]
    DH = D // 2

    EP = _round_up(E, NS * CHUNK * _GRP)
    NP = _round_up(N + 1, NS * CHUNK)

    src = edge_index[0]
    dst = edge_index[1]
    pad = EP - E
    src_p = jnp.concatenate([src, jnp.zeros((pad,), jnp.int32)])
    dst_p = jnp.concatenate([dst, jnp.full((pad,), N, jnp.int32)])
    src2d = src_p.reshape(EP // CHUNK, CHUNK)
    dst2d = dst_p.reshape(EP // CHUNK, CHUNK)

    deg_raw = _make_sc_hist(EP, NP)(dst2d)
    inv_col, gl, gr = _make_tc_prep(N, NP, D)(deg_raw.reshape(NP, 1), x)

    sc_agg = _make_sc_agg(EP, NP, DH)
    tc_mid = _make_tc_round(N, NP, D, final=False)
    tc_fin = _make_tc_round(N, NP, D, final=True)

    for r, W in enumerate((W0, W1, W2)):
        sl_acc, sr_acc = sc_agg(gl, gr, src2d, dst2d)
        if r < 2:
            gl, gr = tc_mid(sl_acc, sr_acc, inv_col, W)
        else:
            (h,) = tc_fin(sl_acc, sr_acc, inv_col, W)
    return h
